# baseline (speedup 1.0000x reference)
import functools

import jax
import jax.numpy as jnp
from jax import lax
from jax.experimental import pallas as pl
from jax.experimental.pallas import tpu as pltpu
from jax.experimental.pallas import tpu_sc as plsc

_NC = 1
_NS = 16
_NW = _NC * _NS

_B = 4096
_L = 200
_E = 64
_N = _B * _L
_CHUNK = 128
_NBUF = 4
_LOOK = 2


def _make_body(nchunk):
    per_w = nchunk * _CHUNK
    nrot = nchunk // _NBUF
    assert nchunk % _NBUF == 0

    def body(ids_hbm, table_hbm, out_hbm, idx_v, rows_v, *sems):
        gsems = sems[:_NBUF]
        ssems = sems[_NBUF:]
        wid = lax.axis_index("s") * _NC + lax.axis_index("c")
        base = wid * per_w

        pltpu.sync_copy(ids_hbm.at[wid], idx_v)

        def gather_start(g, b):
            pltpu.async_copy(table_hbm.at[idx_v.at[g]], rows_v.at[b], gsems[b])

        def gather_wait(g, b):
            pltpu.make_async_copy(
                table_hbm.at[idx_v.at[g]], rows_v.at[b], gsems[b]).wait()

        def store_start(g, b):
            pltpu.async_copy(
                rows_v.at[b], out_hbm.at[pl.ds(base + g * _CHUNK, _CHUNK)],
                ssems[b])

        def store_wait(g, b):
            pltpu.make_async_copy(
                rows_v.at[b], out_hbm.at[pl.ds(base + g * _CHUNK, _CHUNK)],
                ssems[b]).wait()

        for g in range(_LOOK):
            gather_start(g, g % _NBUF)

        for b in range(_NBUF):
            g = b
            if g + _LOOK >= _NBUF:
                store_wait(g + _LOOK - _NBUF, (g + _LOOK) % _NBUF)
            gather_start(g + _LOOK, (g + _LOOK) % _NBUF)
            gather_wait(g, b)
            store_start(g, b)

        def rot(i, carry):
            for b in range(_NBUF):
                g = i * _NBUF + b
                store_wait(g + _LOOK - _NBUF, (b + _LOOK) % _NBUF)
                gather_start(g + _LOOK, (b + _LOOK) % _NBUF)
                gather_wait(g, b)
                store_start(g, b)
            return carry

        lax.fori_loop(1, nrot - 1, rot, 0)

        for b in range(_NBUF):
            g = (nrot - 1) * _NBUF + b
            if g + _LOOK < nchunk:
                store_wait(g + _LOOK - _NBUF, (b + _LOOK) % _NBUF)
                gather_start(g + _LOOK, (b + _LOOK) % _NBUF)
            gather_wait(g, b)
            store_start(g, b)

        for b in range(_NBUF):
            g = (nrot - 1) * _NBUF + b
            store_wait(g, b)

    return body


@functools.lru_cache(maxsize=None)
def _make_emb(nchunk):
    mesh = plsc.VectorSubcoreMesh(core_axis_name="c", subcore_axis_name="s", num_cores=1, num_subcores=16)
    scratch = [
        pltpu.VMEM((nchunk, _CHUNK), jnp.int32),
        pltpu.VMEM((_NBUF, _CHUNK, _E), jnp.float32),
    ] + [pltpu.SemaphoreType.DMA] * (2 * _NBUF)
    return pl.kernel(
        _make_body(nchunk),
        out_type=jax.ShapeDtypeStruct((nchunk * _CHUNK * _NW, _E),
                                      jnp.float32),
        mesh=mesh,
        scratch_types=scratch,
        compiler_params=pltpu.CompilerParams(use_tc_tiling_on_sc=False),
    )


def _emb(ids_flat, table):
    n = ids_flat.shape[0]
    nchunk = n // (_NW * _CHUNK)
    ids3 = ids_flat.reshape(_NW, nchunk, _CHUNK)
    return _make_emb(nchunk)(ids3, table)


@jax.jit
def _run(ids_flat, table):
    return _emb(ids_flat, table)


def kernel(token_ids, table):
    ids_flat = token_ids.reshape(_N).astype(jnp.int32)
    out = _run(ids_flat, table)
    return out.reshape(_B, _L, _E)


# speedup vs baseline: 1.0636x; 1.0636x over previous
import functools

import jax
import jax.numpy as jnp
from jax import lax
from jax.experimental import pallas as pl
from jax.experimental.pallas import tpu as pltpu
from jax.experimental.pallas import tpu_sc as plsc

_NC = 2
_NS = 16
_NW = _NC * _NS

_B = 4096
_L = 200
_E = 64
_N = _B * _L
_CHUNK = 128
_NBUF = 4
_LOOK = 2


def _make_body(nchunk):
    per_w = nchunk * _CHUNK
    nrot = nchunk // _NBUF
    assert nchunk % _NBUF == 0

    def body(ids_hbm, table_hbm, out_hbm, idx_v, rows_v, *sems):
        gsems = sems[:_NBUF]
        ssems = sems[_NBUF:]
        wid = lax.axis_index("s") * _NC + lax.axis_index("c")
        base = wid * per_w

        pltpu.sync_copy(ids_hbm.at[wid], idx_v)

        def gather_start(g, b):
            pltpu.async_copy(table_hbm.at[idx_v.at[g]], rows_v.at[b], gsems[b])

        def gather_wait(g, b):
            pltpu.make_async_copy(
                table_hbm.at[idx_v.at[g]], rows_v.at[b], gsems[b]).wait()

        def store_start(g, b):
            pltpu.async_copy(
                rows_v.at[b], out_hbm.at[pl.ds(base + g * _CHUNK, _CHUNK)],
                ssems[b])

        def store_wait(g, b):
            pltpu.make_async_copy(
                rows_v.at[b], out_hbm.at[pl.ds(base + g * _CHUNK, _CHUNK)],
                ssems[b]).wait()

        for g in range(_LOOK):
            gather_start(g, g % _NBUF)

        for b in range(_NBUF):
            g = b
            if g + _LOOK >= _NBUF:
                store_wait(g + _LOOK - _NBUF, (g + _LOOK) % _NBUF)
            gather_start(g + _LOOK, (g + _LOOK) % _NBUF)
            gather_wait(g, b)
            store_start(g, b)

        def rot(i, carry):
            for b in range(_NBUF):
                g = i * _NBUF + b
                store_wait(g + _LOOK - _NBUF, (b + _LOOK) % _NBUF)
                gather_start(g + _LOOK, (b + _LOOK) % _NBUF)
                gather_wait(g, b)
                store_start(g, b)
            return carry

        lax.fori_loop(1, nrot - 1, rot, 0)

        for b in range(_NBUF):
            g = (nrot - 1) * _NBUF + b
            if g + _LOOK < nchunk:
                store_wait(g + _LOOK - _NBUF, (b + _LOOK) % _NBUF)
                gather_start(g + _LOOK, (b + _LOOK) % _NBUF)
            gather_wait(g, b)
            store_start(g, b)

        for b in range(_NBUF):
            g = (nrot - 1) * _NBUF + b
            store_wait(g, b)

    return body


@functools.lru_cache(maxsize=None)
def _make_emb(nchunk):
    mesh = plsc.VectorSubcoreMesh(core_axis_name="c", subcore_axis_name="s")
    scratch = [
        pltpu.VMEM((nchunk, _CHUNK), jnp.int32),
        pltpu.VMEM((_NBUF, _CHUNK, _E), jnp.float32),
    ] + [pltpu.SemaphoreType.DMA] * (2 * _NBUF)
    return pl.kernel(
        _make_body(nchunk),
        out_type=jax.ShapeDtypeStruct((nchunk * _CHUNK * _NW, _E),
                                      jnp.float32),
        mesh=mesh,
        scratch_types=scratch,
        compiler_params=pltpu.CompilerParams(use_tc_tiling_on_sc=False),
    )


def _emb(ids_flat, table):
    n = ids_flat.shape[0]
    nchunk = n // (_NW * _CHUNK)
    ids3 = ids_flat.reshape(_NW, nchunk, _CHUNK)
    return _make_emb(nchunk)(ids3, table)


@jax.jit
def _run(ids_flat, table):
    return _emb(ids_flat, table)


def kernel(token_ids, table):
    ids_flat = token_ids.reshape(_N).astype(jnp.int32)
    out = _run(ids_flat, table)
    return out.reshape(_B, _L, _E)
